# baseline (speedup 1.0000x reference)
import functools

import jax
import jax.numpy as jnp
from jax import lax
from jax.experimental import pallas as pl
from jax.experimental.pallas import tpu as pltpu
from jax.experimental.pallas import tpu_sc as plsc

IMW = 224
ROWS = 8 * 192 * 224
NC, NS = 2, 16
NW = NC * NS
ROWS_PER_W = ROWS // NW
CHUNK = 128
NCHUNK = ROWS_PER_W // CHUNK
NGRP = IMW // 16

_mesh = plsc.VectorSubcoreMesh(
    core_axis_name="c", subcore_axis_name="s", num_cores=NC, num_subcores=NS)


@functools.partial(
    pl.kernel,
    out_type=jax.ShapeDtypeStruct((ROWS, IMW), jnp.float32),
    mesh=_mesh,
    scratch_types=[
        pltpu.VMEM((IMW,), jnp.int32),
        pltpu.VMEM((CHUNK, IMW), jnp.float32),
        pltpu.VMEM((CHUNK, IMW), jnp.float32),
    ],
    compiler_params=pltpu.CompilerParams(use_tc_tiling_on_sc=False, needs_layout_passes=False),
)
def _sc_flip(x_hbm, idx_hbm, o_hbm, idx_v, in_v, out_v):
    wid = lax.axis_index("s") * NC + lax.axis_index("c")
    pltpu.sync_copy(idx_hbm, idx_v)

    def chunk_body(it, _):
        base = (wid * NCHUNK + it) * CHUNK
        pltpu.sync_copy(x_hbm.at[pl.ds(base, CHUNK)], in_v)
        for g in range(NGRP):
            colvec = idx_v[pl.ds(g * 16, 16)]

            def row_body(r, _, colvec=colvec, g=g):
                rsplat = jnp.full((16,), r, dtype=jnp.int32)
                v = plsc.load_gather(in_v, [rsplat, colvec])
                out_v[r, pl.ds(g * 16, 16)] = v
                return ()

            lax.fori_loop(0, CHUNK, row_body, (), unroll=8)
        pltpu.sync_copy(out_v, o_hbm.at[pl.ds(base, CHUNK)])
        return ()

    lax.fori_loop(0, NCHUNK, chunk_body, ())


def kernel(input, inv_indices):
    x2 = input.reshape(ROWS, IMW)
    out = _sc_flip(x2, inv_indices.astype(jnp.int32))
    return out.reshape(input.shape)


# speedup vs baseline: 1.6723x; 1.6723x over previous
import functools

import jax
import jax.numpy as jnp
from jax import lax
from jax.experimental import pallas as pl
from jax.experimental.pallas import tpu as pltpu
from jax.experimental.pallas import tpu_sc as plsc

IMW = 224
ROWS = 8 * 192 * 224
NC, NS = 2, 16
NW = NC * NS
ROWS_PER_W = ROWS // NW
CHUNK = 128
NCHUNK = ROWS_PER_W // CHUNK
NGRP = IMW // 16

_mesh = plsc.VectorSubcoreMesh(
    core_axis_name="c", subcore_axis_name="s", num_cores=NC, num_subcores=NS)


@functools.partial(
    pl.kernel,
    out_type=jax.ShapeDtypeStruct((ROWS, IMW), jnp.float32),
    mesh=_mesh,
    scratch_types=[
        pltpu.VMEM((IMW,), jnp.int32),
        pltpu.VMEM((CHUNK, IMW), jnp.float32),
        pltpu.VMEM((CHUNK, IMW), jnp.float32),
        pltpu.VMEM((CHUNK, IMW), jnp.float32),
        pltpu.VMEM((CHUNK, IMW), jnp.float32),
        pltpu.SemaphoreType.DMA,
        pltpu.SemaphoreType.DMA,
        pltpu.SemaphoreType.DMA,
        pltpu.SemaphoreType.DMA,
    ],
    compiler_params=pltpu.CompilerParams(
        use_tc_tiling_on_sc=False, needs_layout_passes=False),
)
def _sc_flip(x_hbm, idx_hbm, o_hbm, idx_v, in0, in1, out0, out1,
             in_s0, in_s1, out_s0, out_s1):
    wid = lax.axis_index("s") * NC + lax.axis_index("c")
    first = wid * NCHUNK
    pltpu.sync_copy(idx_hbm, idx_v)
    colvecs = [idx_v[pl.ds(g * 16, 16)] for g in range(NGRP)]

    ins = (in0, in1)
    outs = (out0, out1)
    in_sems = (in_s0, in_s1)
    out_sems = (out_s0, out_s1)

    def start_in(c, b):
        pltpu.async_copy(x_hbm.at[pl.ds((first + c) * CHUNK, CHUNK)],
                         ins[b], in_sems[b])

    def flip_chunk(in_v, out_v):
        @plsc.parallel_loop(0, CHUNK, unroll=4)
        def _(r):
            rsplat = jnp.full((16,), r, dtype=jnp.int32)
            for g in range(NGRP):
                out_v[r, pl.ds(g * 16, 16)] = plsc.load_gather(
                    in_v, [rsplat, colvecs[g]])

    start_in(0, 0)
    start_in(1, 1)

    def pair_body(p, _):
        for b in range(2):
            c = 2 * p + b
            pltpu.make_async_copy(
                x_hbm.at[pl.ds(0, CHUNK)], ins[b], in_sems[b]).wait()

            @pl.when(p > 0)
            def _():
                pltpu.make_async_copy(
                    outs[b], o_hbm.at[pl.ds(0, CHUNK)], out_sems[b]).wait()

            flip_chunk(ins[b], outs[b])
            pltpu.async_copy(outs[b],
                             o_hbm.at[pl.ds((first + c) * CHUNK, CHUNK)],
                             out_sems[b])

            @pl.when(c + 2 < NCHUNK)
            def _():
                start_in(c + 2, b)
        return ()

    lax.fori_loop(0, NCHUNK // 2, pair_body, ())
    for b in range(2):
        pltpu.make_async_copy(
            outs[b], o_hbm.at[pl.ds(0, CHUNK)], out_sems[b]).wait()


def kernel(input, inv_indices):
    x2 = input.reshape(ROWS, IMW)
    out = _sc_flip(x2, inv_indices.astype(jnp.int32))
    return out.reshape(input.shape)


# speedup vs baseline: 7.7195x; 4.6162x over previous
import jax
import jax.numpy as jnp
from jax.experimental import pallas as pl

IMW = 224
ROWS = 8 * 192 * 224
BLOCK_ROWS = 14336


def _flip_body(x_ref, p_ref, o_ref):
    o_ref[...] = jnp.dot(x_ref[...], p_ref[...],
                         preferred_element_type=jnp.float32)


def kernel(input, inv_indices):
    x2 = input.reshape(ROWS, IMW)
    perm = (inv_indices[None, :].astype(jnp.int32)
            == jnp.arange(IMW, dtype=jnp.int32)[:, None]).astype(jnp.float32)
    out = pl.pallas_call(
        _flip_body,
        grid=(ROWS // BLOCK_ROWS,),
        in_specs=[
            pl.BlockSpec((BLOCK_ROWS, IMW), lambda i: (i, 0)),
            pl.BlockSpec((IMW, IMW), lambda i: (0, 0)),
        ],
        out_specs=pl.BlockSpec((BLOCK_ROWS, IMW), lambda i: (i, 0)),
        out_shape=jax.ShapeDtypeStruct((ROWS, IMW), input.dtype),
    )(x2, perm)
    return out.reshape(input.shape)
